# Initial kernel scaffold; baseline (speedup 1.0000x reference)
import jax
import jax.numpy as jnp
from jax.experimental import pallas as pl


def kernel(layer_input, W_router, W_in, W_out):
    raise NotImplementedError("write your pallas kernel here")


# speedup vs baseline: 1.3402x; 1.3402x over previous
import functools

import jax
import jax.numpy as jnp
from jax.experimental import pallas as pl
from jax.experimental.pallas import tpu as pltpu

_NUM_EXPERTS = 8
_TOP_K = 2
_ROW_TILE = 256


def _router_kernel(x_ref, wr_ref, idx_ref, w_ref):
    x = x_ref[...]
    wr = wr_ref[...]
    logits = jax.lax.dot_general(
        wr, x, (((1,), (1,)), ((), ())), preferred_element_type=jnp.float32
    )
    neg_inf = jnp.float32(-jnp.inf)
    e1 = jnp.argmax(logits, axis=0).astype(jnp.int32)
    m1 = jnp.max(logits, axis=0)
    rows = jax.lax.broadcasted_iota(jnp.int32, logits.shape, 0)
    masked = jnp.where(rows == e1[None, :], neg_inf, logits)
    e2 = jnp.argmax(masked, axis=0).astype(jnp.int32)
    m2 = jnp.max(masked, axis=0)
    t = jnp.exp(m2 - m1)
    w_ref[...] = jnp.stack([1.0 / (1.0 + t), t / (1.0 + t)], axis=0)
    idx_ref[...] = jnp.stack([e1, e2], axis=0)


def _router(x, w_router):
    T = x.shape[0]
    return pl.pallas_call(
        _router_kernel,
        out_shape=[
            jax.ShapeDtypeStruct((2, T), jnp.int32),
            jax.ShapeDtypeStruct((2, T), jnp.float32),
        ],
    )(x, w_router)


def _gmm_kernel(tile_ref, exp_ref, fv_ref, x_ref, scale_ref, win_ref,
                wout_ref, out_ref):
    i = pl.program_id(0)
    x = x_ref[...]
    scale = scale_ref[0, 0, :]
    gh = jnp.dot(x, win_ref[0], preferred_element_type=jnp.float32)
    ff = gh.shape[1] // 2
    gate = gh[:, :ff]
    up = gh[:, ff:]
    h = gate * jax.lax.logistic(gate) * up
    y = jnp.dot(h, wout_ref[0], preferred_element_type=jnp.float32)
    y = y * scale[:, None]

    @pl.when(fv_ref[i] == 1)
    def _init():
        out_ref[...] = jnp.zeros_like(out_ref)

    out_ref[...] += y


def _grouped_mlp(x_sorted, scale3, w_in, w_out, tile_ids, exp_ids, fv):
    m, d = x_sorted.shape
    e, _, ff2 = w_in.shape
    ff = ff2 // 2
    b = _ROW_TILE
    g_max = tile_ids.shape[0]
    grid_spec = pltpu.PrefetchScalarGridSpec(
        num_scalar_prefetch=3,
        grid=(g_max,),
        in_specs=[
            pl.BlockSpec((b, d), lambda i, t, ex, f: (t[i], 0)),
            pl.BlockSpec((1, 1, b), lambda i, t, ex, f: (i, 0, 0)),
            pl.BlockSpec((1, d, ff2), lambda i, t, ex, f: (ex[i], 0, 0)),
            pl.BlockSpec((1, ff, d), lambda i, t, ex, f: (ex[i], 0, 0)),
        ],
        out_specs=pl.BlockSpec((b, d), lambda i, t, ex, f: (t[i], 0)),
    )
    return pl.pallas_call(
        _gmm_kernel,
        grid_spec=grid_spec,
        out_shape=jax.ShapeDtypeStruct((m, d), jnp.float32),
        compiler_params=pltpu.CompilerParams(
            dimension_semantics=("arbitrary",)),
    )(tile_ids, exp_ids, fv, x_sorted, scale3, w_in, w_out)


def kernel(layer_input, W_router, W_in, W_out):
    bsz, seq, d = layer_input.shape
    x = layer_input.reshape(-1, d)
    T = x.shape[0]
    M = _TOP_K * T
    B = _ROW_TILE
    num_tiles = M // B
    g_max = num_tiles + _NUM_EXPERTS - 1

    idx2, w2 = _router(x, W_router)
    e_flat = idx2.T.reshape(-1)
    w_flat = w2.T.reshape(-1)

    order = jnp.argsort(e_flat, stable=True).astype(jnp.int32)
    tok = order // _TOP_K
    e_sorted = e_flat[order]
    w_sorted = w_flat[order]
    inv = jnp.zeros((M,), jnp.int32).at[order].set(
        jnp.arange(M, dtype=jnp.int32))
    pos = inv.reshape(T, _TOP_K)

    sizes = jnp.bincount(e_flat, length=_NUM_EXPERTS).astype(jnp.int32)
    ends = jnp.cumsum(sizes)
    starts = ends - sizes
    first_tile = starts // B
    last_tile = jnp.where(sizes > 0, (ends - 1) // B, first_tile)
    tiles_g = jnp.where(sizes > 0, last_tile - first_tile + 1, 0)
    wends = jnp.cumsum(tiles_g)
    wstart = wends - tiles_g
    total = wends[-1]
    iarr = jnp.arange(g_max, dtype=jnp.int32)
    g = (jnp.searchsorted(wstart, iarr, side="right") - 1).astype(jnp.int32)
    g = jnp.clip(g, 0, _NUM_EXPERTS - 1)
    valid = iarr < total
    tile_ids = jnp.where(
        valid,
        jnp.clip(first_tile[g] + (iarr - wstart[g]), 0, num_tiles - 1),
        num_tiles - 1,
    ).astype(jnp.int32)
    exp_ids = g
    fv = jnp.concatenate(
        [jnp.ones((1,), jnp.int32),
         (tile_ids[1:] != tile_ids[:-1]).astype(jnp.int32)])

    row_idx = tile_ids[:, None] * B + jnp.arange(B, dtype=jnp.int32)[None, :]
    scale = jnp.where(
        (e_sorted[row_idx] == exp_ids[:, None]) & valid[:, None],
        w_sorted[row_idx], 0.0).astype(jnp.float32)
    scale3 = scale.reshape(g_max, 1, B)

    x_sorted = x[tok]
    y_sorted = _grouped_mlp(x_sorted, scale3, W_in, W_out,
                            tile_ids, exp_ids, fv)
    out = y_sorted[pos[:, 0]] + y_sorted[pos[:, 1]]
    return out.reshape(bsz, seq, d)
